# Initial kernel scaffold; baseline (speedup 1.0000x reference)
import jax
import jax.numpy as jnp
from jax.experimental import pallas as pl


def kernel(x, edge_attr, u, edge_index, W_e, b_e, W_v, b_v, W_u, b_u):
    raise NotImplementedError("write your pallas kernel here")


# speedup vs baseline: 3.2811x; 3.2811x over previous
import functools

import jax
import jax.numpy as jnp
from jax import lax
from jax.experimental import pallas as pl
from jax.experimental.pallas import tpu as pltpu
from jax.experimental.pallas import tpu_sc as plsc

N_NODES = 10000
N_EDGES = 320000
D = 128
D_E_IN = 16

NC = 2
NS = 16
NW = NC * NS
EPT = N_EDGES // NW
CH = 80
NCHUNK = EPT // CH
N_PAD = 10240
RPT = N_PAD // NS


def _proj_body(x_ref, wr_ref, ws_ref, u_ref, weu_ref, be_ref,
               xr_ref, xs_ref, ec_ref):
    xr_ref[...] = jnp.dot(x_ref[...], wr_ref[...],
                          preferred_element_type=jnp.float32)
    xs_ref[...] = jnp.dot(x_ref[...], ws_ref[...],
                          preferred_element_type=jnp.float32)
    ec_ref[...] = jnp.dot(u_ref[...], weu_ref[...],
                          preferred_element_type=jnp.float32) + be_ref[...]


def _proj(x, We_r, We_s, u2, We_u, be2):
    return pl.pallas_call(
        _proj_body,
        out_shape=(
            jax.ShapeDtypeStruct((N_NODES, D), jnp.float32),
            jax.ShapeDtypeStruct((N_NODES, D), jnp.float32),
            jax.ShapeDtypeStruct((1, D), jnp.float32),
        ),
    )(x, We_r, We_s, u2, We_u, be2)


_EB = 2000


def _edge_body(ea_ref, w_ref, ec_ref, o_ref):
    o_ref[...] = jnp.dot(ea_ref[...], w_ref[...],
                         preferred_element_type=jnp.float32) + ec_ref[...]


def _edge_mm(edge_attr, We_e, ec):
    return pl.pallas_call(
        _edge_body,
        grid=(N_EDGES // _EB,),
        in_specs=[
            pl.BlockSpec((_EB, D_E_IN), lambda i: (i, 0)),
            pl.BlockSpec((D_E_IN, D), lambda i: (0, 0)),
            pl.BlockSpec((1, D), lambda i: (0, 0)),
        ],
        out_specs=pl.BlockSpec((_EB, D), lambda i: (i, 0)),
        out_shape=jax.ShapeDtypeStruct((N_EDGES, D), jnp.float32),
    )(edge_attr, We_e, ec)


def _sc_body(a_hbm, dst_hbm, src_hbm, xr_hbm, xs_hbm, zeros_hbm,
             e_hbm, aggr_hbm,
             dst_v, src_v, a_v, xr_v, xs_v, aggr_sh, sem1, sem2):
    cid = lax.axis_index("c")
    sid = lax.axis_index("s")
    wid = sid * NC + cid

    pltpu.sync_copy(zeros_hbm.at[pl.ds(sid * RPT, RPT)],
                    aggr_sh.at[pl.ds(sid * RPT, RPT)])
    plsc.subcore_barrier()

    ebase = wid * EPT

    def chunk(k, carry):
        base = ebase + k * CH
        pltpu.sync_copy(dst_hbm.at[pl.ds(base, CH)], dst_v)
        pltpu.sync_copy(src_hbm.at[pl.ds(base, CH)], src_v)
        g1 = pltpu.async_copy(xr_hbm.at[dst_v], xr_v, sem1)
        g2 = pltpu.async_copy(xs_hbm.at[src_v], xs_v, sem2)
        pltpu.sync_copy(a_hbm.at[pl.ds(base, CH)], a_v)
        g1.wait()
        g2.wait()

        def row(i, c2):
            for j in range(D // 16):
                sl = pl.ds(j * 16, 16)
                v = a_v[i, sl] + xr_v[i, sl] + xs_v[i, sl]
                a_v[i, sl] = jnp.maximum(v, 0.0)
            return c2

        lax.fori_loop(0, CH, row, 0)
        pltpu.sync_copy(a_v, e_hbm.at[pl.ds(base, CH)])
        pltpu.sync_copy(a_v, aggr_sh.at[dst_v], add=True)
        return carry

    lax.fori_loop(0, NCHUNK, chunk, 0)
    plsc.subcore_barrier()
    pltpu.sync_copy(aggr_sh.at[pl.ds(sid * RPT, RPT)],
                    aggr_hbm.at[cid, pl.ds(sid * RPT, RPT)])


def _sc_edges(A, dst, src, Xr, Xs, zeros):
    mesh = plsc.VectorSubcoreMesh(core_axis_name="c", subcore_axis_name="s")
    fn = functools.partial(
        pl.kernel,
        mesh=mesh,
        out_type=(
            jax.ShapeDtypeStruct((N_EDGES, D), jnp.float32),
            jax.ShapeDtypeStruct((NC, N_PAD, D), jnp.float32),
        ),
        scratch_types=[
            pltpu.VMEM((CH,), jnp.int32),
            pltpu.VMEM((CH,), jnp.int32),
            pltpu.VMEM((CH, D), jnp.float32),
            pltpu.VMEM((CH, D), jnp.float32),
            pltpu.VMEM((CH, D), jnp.float32),
            pltpu.VMEM_SHARED((N_PAD, D), jnp.float32),
            pltpu.SemaphoreType.DMA,
            pltpu.SemaphoreType.DMA,
        ],
    )(_sc_body)
    return fn(A, dst, src, Xr, Xs, zeros)


def _node_body(ag_ref, x_ref, u_ref, wva_ref, wvx_ref, wvu_ref, bv_ref,
               wue_ref, wuv_ref, wuu_ref, bu_ref, v_ref, un_ref):
    aggr = ag_ref[0, :N_NODES] + ag_ref[1, :N_NODES]
    cv = jnp.dot(u_ref[...], wvu_ref[...],
                 preferred_element_type=jnp.float32) + bv_ref[...]
    v = jnp.maximum(
        jnp.dot(aggr, wva_ref[...], preferred_element_type=jnp.float32)
        + jnp.dot(x_ref[...], wvx_ref[...], preferred_element_type=jnp.float32)
        + cv, 0.0)
    v_ref[...] = v
    ae = jnp.sum(aggr, axis=0, keepdims=True) * (1.0 / N_EDGES)
    av = jnp.sum(v, axis=0, keepdims=True) * (1.0 / N_NODES)
    un = (jnp.dot(ae, wue_ref[...], preferred_element_type=jnp.float32)
          + jnp.dot(av, wuv_ref[...], preferred_element_type=jnp.float32)
          + jnp.dot(u_ref[...], wuu_ref[...], preferred_element_type=jnp.float32)
          + bu_ref[...])
    un_ref[...] = jnp.maximum(un, 0.0)


def _node(aggr2, x, u2, W_v, b_v, W_u, b_u):
    return pl.pallas_call(
        _node_body,
        out_shape=(
            jax.ShapeDtypeStruct((N_NODES, D), jnp.float32),
            jax.ShapeDtypeStruct((1, D), jnp.float32),
        ),
    )(aggr2, x, u2, W_v[:D], W_v[D:2 * D], W_v[2 * D:3 * D],
      b_v.reshape(1, D), W_u[:D], W_u[D:2 * D], W_u[2 * D:3 * D],
      b_u.reshape(1, D))


def kernel(x, edge_attr, u, edge_index, W_e, b_e, W_v, b_v, W_u, b_u):
    ei = edge_index.astype(jnp.int32)
    src = ei[0]
    dst = ei[1]
    We_e = W_e[:D_E_IN]
    We_r = W_e[D_E_IN:D_E_IN + D]
    We_s = W_e[D_E_IN + D:D_E_IN + 2 * D]
    We_u = W_e[D_E_IN + 2 * D:]
    u2 = u.reshape(1, D)
    be2 = b_e.reshape(1, D)

    Xr, Xs, ec = _proj(x, We_r, We_s, u2, We_u, be2)
    A = _edge_mm(edge_attr, We_e, ec)
    zeros = jnp.zeros((N_PAD, D), jnp.float32)
    e_new, aggr2 = _sc_edges(A, dst, src, Xr, Xs, zeros)
    v_new, u_new2 = _node(aggr2, x, u2, W_v, b_v, W_u, b_u)
    return (e_new, v_new, u_new2.reshape(D))


# speedup vs baseline: 5.1581x; 1.5721x over previous
import functools

import jax
import jax.numpy as jnp
from jax import lax
from jax.experimental import pallas as pl
from jax.experimental.pallas import tpu as pltpu
from jax.experimental.pallas import tpu_sc as plsc

N_NODES = 10000
N_EDGES = 320000
D = 128
D_E_IN = 16

NC = 2
NS = 16
NW = NC * NS
EPT = N_EDGES // NW
CH = 40
NCHUNK = EPT // CH
N_PAD = 10112
RPT = N_PAD // NS


def _proj_body(x_ref, wr_ref, ws_ref, u_ref, weu_ref, be_ref,
               xr_ref, xs_ref, ec_ref):
    xr_ref[...] = jnp.dot(x_ref[...], wr_ref[...],
                          preferred_element_type=jnp.float32)
    xs_ref[...] = jnp.dot(x_ref[...], ws_ref[...],
                          preferred_element_type=jnp.float32)
    ec_ref[...] = jnp.dot(u_ref[...], weu_ref[...],
                          preferred_element_type=jnp.float32) + be_ref[...]


def _proj(x, We_r, We_s, u2, We_u, be2):
    return pl.pallas_call(
        _proj_body,
        out_shape=(
            jax.ShapeDtypeStruct((N_NODES, D), jnp.float32),
            jax.ShapeDtypeStruct((N_NODES, D), jnp.float32),
            jax.ShapeDtypeStruct((1, D), jnp.float32),
        ),
    )(x, We_r, We_s, u2, We_u, be2)


_EB = 2000


def _edge_body(ea_ref, w_ref, ec_ref, o_ref):
    o_ref[...] = jnp.dot(ea_ref[...], w_ref[...],
                         preferred_element_type=jnp.float32) + ec_ref[...]


def _edge_mm(edge_attr, We_e, ec):
    return pl.pallas_call(
        _edge_body,
        grid=(N_EDGES // _EB,),
        in_specs=[
            pl.BlockSpec((_EB, D_E_IN), lambda i: (i, 0)),
            pl.BlockSpec((D_E_IN, D), lambda i: (0, 0)),
            pl.BlockSpec((1, D), lambda i: (0, 0)),
        ],
        out_specs=pl.BlockSpec((_EB, D), lambda i: (i, 0)),
        out_shape=jax.ShapeDtypeStruct((N_EDGES, D), jnp.float32),
    )(edge_attr, We_e, ec)


class _Slot:
    def __init__(self, gd, gs, sd, xr, xs, ab, gi, si, r, s, a, st, sc):
        self.gd = gd
        self.gs = gs
        self.sd = sd
        self.xr = xr
        self.xs = xs
        self.ab = ab
        self.gi = gi
        self.si = si
        self.r = r
        self.s = s
        self.a = a
        self.st = st
        self.sc = sc


def _sc_body(a_hbm, dst_hbm, src_hbm, xr_hbm, xs_hbm, zeros_hbm,
             e_hbm, aggr_hbm, *rest):
    cid = lax.axis_index("c")
    sid = lax.axis_index("s")
    wid = sid * NC + cid
    ebase = wid * EPT

    bufs, sems, aggr_sh = rest[:18], rest[19:], rest[18]
    SLOT = tuple(_Slot(*bufs[6 * i:6 * i + 6], *sems[7 * i:7 * i + 7])
                 for i in range(3))

    pltpu.sync_copy(zeros_hbm.at[pl.ds(sid * RPT, RPT)],
                    aggr_sh.at[pl.ds(sid * RPT, RPT)])
    plsc.subcore_barrier()

    def issue_gidx(k, S):
        base = ebase + k * CH
        pltpu.async_copy(dst_hbm.at[pl.ds(base, CH)], S.gd, S.gi)
        pltpu.async_copy(src_hbm.at[pl.ds(base, CH)], S.gs, S.gi)

    def wait_gidx(S):
        dm = dst_hbm.at[pl.ds(0, CH)]
        pltpu.make_async_copy(dm, S.gd, S.gi).wait()
        pltpu.make_async_copy(dm, S.gs, S.gi).wait()

    def issue_sidx(k, S):
        pltpu.async_copy(dst_hbm.at[pl.ds(ebase + k * CH, CH)], S.sd, S.si)

    def issue_gathers(k, S):
        pltpu.async_copy(xr_hbm.at[S.gd], S.xr, S.r)
        pltpu.async_copy(xs_hbm.at[S.gs], S.xs, S.s)
        pltpu.async_copy(a_hbm.at[pl.ds(ebase + k * CH, CH)], S.ab, S.a)

    def wait_in(S):
        dm = a_hbm.at[pl.ds(0, CH)]
        pltpu.make_async_copy(dm, S.xr, S.r).wait()
        pltpu.make_async_copy(dm, S.xs, S.s).wait()
        pltpu.make_async_copy(dm, S.ab, S.a).wait()

    def compute(S):
        def row(i, c2):
            for j in range(D // 16):
                sl = pl.ds(j * 16, 16)
                v = S.ab[i, sl] + S.xr[i, sl] + S.xs[i, sl]
                S.ab[i, sl] = jnp.maximum(v, 0.0)
            return c2

        lax.fori_loop(0, CH, row, 0)

    def issue_out(k, S):
        pltpu.make_async_copy(dst_hbm.at[pl.ds(0, CH)], S.sd, S.si).wait()
        pltpu.async_copy(S.ab, e_hbm.at[pl.ds(ebase + k * CH, CH)], S.st)
        pltpu.async_copy(S.ab, aggr_sh.at[S.sd], S.sc, add=True)

    def wait_out(S):
        dm = a_hbm.at[pl.ds(0, CH)]
        pltpu.make_async_copy(dm, S.xr, S.st).wait()
        pltpu.make_async_copy(dm, S.xr, S.sc).wait()

    def step(k, cur, nxt, first=False):
        wait_in(cur)

        @pl.when(k + 3 < NCHUNK)
        def _():
            issue_gidx(k + 3, cur)

        compute(cur)
        issue_out(k, cur)

        @pl.when(k + 2 < NCHUNK)
        def _():
            if not first:
                wait_out(nxt)
                issue_sidx(k + 2, nxt)
            wait_gidx(nxt)
            issue_gathers(k + 2, nxt)

    issue_gidx(0, SLOT[0])
    issue_gidx(1, SLOT[1])
    issue_gidx(2, SLOT[2])
    issue_sidx(0, SLOT[0])
    issue_sidx(1, SLOT[1])
    issue_sidx(2, SLOT[2])
    wait_gidx(SLOT[0])
    issue_gathers(0, SLOT[0])
    wait_gidx(SLOT[1])
    issue_gathers(1, SLOT[1])

    step(0, SLOT[0], SLOT[2], first=True)
    step(1, SLOT[1], SLOT[0])

    def grp(g, carry):
        k = 3 * g + 2
        step(k, SLOT[2], SLOT[1])
        step(k + 1, SLOT[0], SLOT[2])
        step(k + 2, SLOT[1], SLOT[0])
        return carry

    lax.fori_loop(0, (NCHUNK - 2) // 3, grp, 0)
    step(NCHUNK - 2, SLOT[2], SLOT[1])
    step(NCHUNK - 1, SLOT[0], SLOT[2])

    wait_out(SLOT[0])
    wait_out(SLOT[1])
    wait_out(SLOT[2])
    plsc.subcore_barrier()
    pltpu.sync_copy(aggr_sh.at[pl.ds(sid * RPT, RPT)],
                    aggr_hbm.at[cid, pl.ds(sid * RPT, RPT)])


def _sc_edges(A, dst, src, Xr, Xs, zeros):
    mesh = plsc.VectorSubcoreMesh(core_axis_name="c", subcore_axis_name="s")
    slot_bufs = []
    for _ in range(3):
        slot_bufs += [pltpu.VMEM((CH,), jnp.int32)] * 3
        slot_bufs += [pltpu.VMEM((CH, D), jnp.float32)] * 3
    fn = functools.partial(
        pl.kernel,
        mesh=mesh,
        out_type=(
            jax.ShapeDtypeStruct((N_EDGES, D), jnp.float32),
            jax.ShapeDtypeStruct((NC, N_PAD, D), jnp.float32),
        ),
        scratch_types=slot_bufs + [
            pltpu.VMEM_SHARED((N_PAD, D), jnp.float32),
        ] + [pltpu.SemaphoreType.DMA] * 21,
    )(_sc_body)
    return fn(A, dst, src, Xr, Xs, zeros)


def _node_body(ag_ref, x_ref, u_ref, wva_ref, wvx_ref, wvu_ref, bv_ref,
               wue_ref, wuv_ref, wuu_ref, bu_ref, v_ref, un_ref):
    aggr = ag_ref[0, :N_NODES] + ag_ref[1, :N_NODES]
    cv = jnp.dot(u_ref[...], wvu_ref[...],
                 preferred_element_type=jnp.float32) + bv_ref[...]
    v = jnp.maximum(
        jnp.dot(aggr, wva_ref[...], preferred_element_type=jnp.float32)
        + jnp.dot(x_ref[...], wvx_ref[...], preferred_element_type=jnp.float32)
        + cv, 0.0)
    v_ref[...] = v
    ae = jnp.sum(aggr, axis=0, keepdims=True) * (1.0 / N_EDGES)
    av = jnp.sum(v, axis=0, keepdims=True) * (1.0 / N_NODES)
    un = (jnp.dot(ae, wue_ref[...], preferred_element_type=jnp.float32)
          + jnp.dot(av, wuv_ref[...], preferred_element_type=jnp.float32)
          + jnp.dot(u_ref[...], wuu_ref[...], preferred_element_type=jnp.float32)
          + bu_ref[...])
    un_ref[...] = jnp.maximum(un, 0.0)


def _node(aggr2, x, u2, W_v, b_v, W_u, b_u):
    return pl.pallas_call(
        _node_body,
        out_shape=(
            jax.ShapeDtypeStruct((N_NODES, D), jnp.float32),
            jax.ShapeDtypeStruct((1, D), jnp.float32),
        ),
    )(aggr2, x, u2, W_v[:D], W_v[D:2 * D], W_v[2 * D:3 * D],
      b_v.reshape(1, D), W_u[:D], W_u[D:2 * D], W_u[2 * D:3 * D],
      b_u.reshape(1, D))


def kernel(x, edge_attr, u, edge_index, W_e, b_e, W_v, b_v, W_u, b_u):
    ei = edge_index.astype(jnp.int32)
    src = ei[0]
    dst = ei[1]
    We_e = W_e[:D_E_IN]
    We_r = W_e[D_E_IN:D_E_IN + D]
    We_s = W_e[D_E_IN + D:D_E_IN + 2 * D]
    We_u = W_e[D_E_IN + 2 * D:]
    u2 = u.reshape(1, D)
    be2 = b_e.reshape(1, D)

    Xr, Xs, ec = _proj(x, We_r, We_s, u2, We_u, be2)
    A = _edge_mm(edge_attr, We_e, ec)
    zeros = jnp.zeros((N_PAD, D), jnp.float32)
    e_new, aggr2 = _sc_edges(A, dst, src, Xr, Xs, zeros)
    v_new, u_new2 = _node(aggr2, x, u2, W_v, b_v, W_u, b_u)
    return (e_new, v_new, u_new2.reshape(D))


# speedup vs baseline: 7.2949x; 1.4143x over previous
import functools

import jax
import jax.numpy as jnp
from jax import lax
from jax.experimental import pallas as pl
from jax.experimental.pallas import tpu as pltpu
from jax.experimental.pallas import tpu_sc as plsc

N_NODES = 10000
N_EDGES = 320000
D = 128
D_E_IN = 16

NC = 2
NS = 16
NW = NC * NS
EPT = N_EDGES // NW
CH = 40
NCHUNK = EPT // CH
N_PAD = 10112
RPT = N_PAD // NS


def _proj_body(x_ref, wr_ref, ws_ref, u_ref, weu_ref, be_ref,
               xr_ref, xs_ref, ec_ref):
    xr_ref[...] = jnp.dot(x_ref[...], wr_ref[...],
                          preferred_element_type=jnp.float32)
    xs_ref[...] = jnp.dot(x_ref[...], ws_ref[...],
                          preferred_element_type=jnp.float32)
    ec_ref[...] = jnp.dot(u_ref[...], weu_ref[...],
                          preferred_element_type=jnp.float32) + be_ref[...]


def _proj(x, We_r, We_s, u2, We_u, be2):
    return pl.pallas_call(
        _proj_body,
        out_shape=(
            jax.ShapeDtypeStruct((N_NODES, D), jnp.float32),
            jax.ShapeDtypeStruct((N_NODES, D), jnp.float32),
            jax.ShapeDtypeStruct((1, D), jnp.float32),
        ),
    )(x, We_r, We_s, u2, We_u, be2)


_EB = 6400


def _edge_body(eat_ref, w_ref, ec_ref, o_ref):
    o_ref[...] = lax.dot_general(
        eat_ref[...], w_ref[...], (((0,), (0,)), ((), ())),
        preferred_element_type=jnp.float32) + ec_ref[...]


def _edge_mm(edge_attr_t, We_e, ec):
    return pl.pallas_call(
        _edge_body,
        grid=(N_EDGES // _EB,),
        in_specs=[
            pl.BlockSpec((D_E_IN, _EB), lambda i: (0, i)),
            pl.BlockSpec((D_E_IN, D), lambda i: (0, 0)),
            pl.BlockSpec((1, D), lambda i: (0, 0)),
        ],
        out_specs=pl.BlockSpec((_EB, D), lambda i: (i, 0)),
        out_shape=jax.ShapeDtypeStruct((N_EDGES, D), jnp.float32),
    )(edge_attr_t, We_e, ec)


class _Slot:
    def __init__(self, gd, gs, sd, xr, xs, ab, gi, si, r, s, a, st, sc):
        self.gd = gd
        self.gs = gs
        self.sd = sd
        self.xr = xr
        self.xs = xs
        self.ab = ab
        self.gi = gi
        self.si = si
        self.r = r
        self.s = s
        self.a = a
        self.st = st
        self.sc = sc


def _sc_body(a_hbm, dst_hbm, src_hbm, xr_hbm, xs_hbm, zeros_hbm,
             e_hbm, aggr_hbm, *rest):
    cid = lax.axis_index("c")
    sid = lax.axis_index("s")
    wid = sid * NC + cid
    ebase = wid * EPT

    bufs, sems, aggr_sh = rest[:18], rest[19:], rest[18]
    SLOT = tuple(_Slot(*bufs[6 * i:6 * i + 6], *sems[7 * i:7 * i + 7])
                 for i in range(3))

    pltpu.sync_copy(zeros_hbm.at[pl.ds(sid * RPT, RPT)],
                    aggr_sh.at[pl.ds(sid * RPT, RPT)])
    plsc.subcore_barrier()

    def issue_gidx(k, S):
        base = ebase + k * CH
        pltpu.async_copy(dst_hbm.at[pl.ds(base, CH)], S.gd, S.gi)
        pltpu.async_copy(src_hbm.at[pl.ds(base, CH)], S.gs, S.gi)

    def wait_gidx(S):
        dm = dst_hbm.at[pl.ds(0, CH)]
        pltpu.make_async_copy(dm, S.gd, S.gi).wait()
        pltpu.make_async_copy(dm, S.gs, S.gi).wait()

    def issue_sidx(k, S):
        pltpu.async_copy(dst_hbm.at[pl.ds(ebase + k * CH, CH)], S.sd, S.si)

    def issue_gathers(k, S):
        pltpu.async_copy(xr_hbm.at[S.gd], S.xr, S.r)
        pltpu.async_copy(xs_hbm.at[S.gs], S.xs, S.s)
        pltpu.async_copy(a_hbm.at[pl.ds(ebase + k * CH, CH)], S.ab, S.a)

    def wait_in(S):
        dm = a_hbm.at[pl.ds(0, CH)]
        pltpu.make_async_copy(dm, S.xr, S.r).wait()
        pltpu.make_async_copy(dm, S.xs, S.s).wait()
        pltpu.make_async_copy(dm, S.ab, S.a).wait()

    def compute(S):
        def row(i, c2):
            for j in range(D // 16):
                sl = pl.ds(j * 16, 16)
                v = S.ab[i, sl] + S.xr[i, sl] + S.xs[i, sl]
                S.ab[i, sl] = jnp.maximum(v, 0.0)
            return c2

        lax.fori_loop(0, CH, row, 0)

    def issue_out(k, S):
        pltpu.make_async_copy(dst_hbm.at[pl.ds(0, CH)], S.sd, S.si).wait()
        pltpu.async_copy(S.ab, e_hbm.at[pl.ds(ebase + k * CH, CH)], S.st)
        pltpu.async_copy(S.ab, aggr_sh.at[S.sd], S.sc, add=True)

    def wait_out(S):
        dm = a_hbm.at[pl.ds(0, CH)]
        pltpu.make_async_copy(dm, S.xr, S.st).wait()
        pltpu.make_async_copy(dm, S.xr, S.sc).wait()

    def step(k, cur, nxt, first=False):
        wait_in(cur)

        @pl.when(k + 3 < NCHUNK)
        def _():
            issue_gidx(k + 3, cur)

        compute(cur)
        issue_out(k, cur)

        @pl.when(k + 2 < NCHUNK)
        def _():
            if not first:
                wait_out(nxt)
                issue_sidx(k + 2, nxt)
            wait_gidx(nxt)
            issue_gathers(k + 2, nxt)

    issue_gidx(0, SLOT[0])
    issue_gidx(1, SLOT[1])
    issue_gidx(2, SLOT[2])
    issue_sidx(0, SLOT[0])
    issue_sidx(1, SLOT[1])
    issue_sidx(2, SLOT[2])
    wait_gidx(SLOT[0])
    issue_gathers(0, SLOT[0])
    wait_gidx(SLOT[1])
    issue_gathers(1, SLOT[1])

    step(0, SLOT[0], SLOT[2], first=True)
    step(1, SLOT[1], SLOT[0])

    def grp(g, carry):
        k = 3 * g + 2
        step(k, SLOT[2], SLOT[1])
        step(k + 1, SLOT[0], SLOT[2])
        step(k + 2, SLOT[1], SLOT[0])
        return carry

    lax.fori_loop(0, (NCHUNK - 2) // 3, grp, 0)
    step(NCHUNK - 2, SLOT[2], SLOT[1])
    step(NCHUNK - 1, SLOT[0], SLOT[2])

    wait_out(SLOT[0])
    wait_out(SLOT[1])
    wait_out(SLOT[2])
    plsc.subcore_barrier()
    pltpu.sync_copy(aggr_sh.at[pl.ds(sid * RPT, RPT)],
                    aggr_hbm.at[cid, pl.ds(sid * RPT, RPT)])


def _sc_edges(A, dst, src, Xr, Xs, zeros):
    mesh = plsc.VectorSubcoreMesh(core_axis_name="c", subcore_axis_name="s")
    slot_bufs = []
    for _ in range(3):
        slot_bufs += [pltpu.VMEM((CH,), jnp.int32)] * 3
        slot_bufs += [pltpu.VMEM((CH, D), jnp.float32)] * 3
    fn = functools.partial(
        pl.kernel,
        mesh=mesh,
        out_type=(
            jax.ShapeDtypeStruct((N_EDGES, D), jnp.float32),
            jax.ShapeDtypeStruct((NC, N_PAD, D), jnp.float32),
        ),
        scratch_types=slot_bufs + [
            pltpu.VMEM_SHARED((N_PAD, D), jnp.float32),
        ] + [pltpu.SemaphoreType.DMA] * 21,
    )(_sc_body)
    return fn(A, dst, src, Xr, Xs, zeros)


def _node_body(ag_ref, x_ref, u_ref, wva_ref, wvx_ref, wvu_ref, bv_ref,
               wue_ref, wuv_ref, wuu_ref, bu_ref, v_ref, un_ref):
    aggr = ag_ref[0, :N_NODES] + ag_ref[1, :N_NODES]
    cv = jnp.dot(u_ref[...], wvu_ref[...],
                 preferred_element_type=jnp.float32) + bv_ref[...]
    v = jnp.maximum(
        jnp.dot(aggr, wva_ref[...], preferred_element_type=jnp.float32)
        + jnp.dot(x_ref[...], wvx_ref[...], preferred_element_type=jnp.float32)
        + cv, 0.0)
    v_ref[...] = v
    ae = jnp.sum(aggr, axis=0, keepdims=True) * (1.0 / N_EDGES)
    av = jnp.sum(v, axis=0, keepdims=True) * (1.0 / N_NODES)
    un = (jnp.dot(ae, wue_ref[...], preferred_element_type=jnp.float32)
          + jnp.dot(av, wuv_ref[...], preferred_element_type=jnp.float32)
          + jnp.dot(u_ref[...], wuu_ref[...], preferred_element_type=jnp.float32)
          + bu_ref[...])
    un_ref[...] = jnp.maximum(un, 0.0)


def _node(aggr2, x, u2, W_v, b_v, W_u, b_u):
    return pl.pallas_call(
        _node_body,
        out_shape=(
            jax.ShapeDtypeStruct((N_NODES, D), jnp.float32),
            jax.ShapeDtypeStruct((1, D), jnp.float32),
        ),
    )(aggr2, x, u2, W_v[:D], W_v[D:2 * D], W_v[2 * D:3 * D],
      b_v.reshape(1, D), W_u[:D], W_u[D:2 * D], W_u[2 * D:3 * D],
      b_u.reshape(1, D))


def kernel(x, edge_attr, u, edge_index, W_e, b_e, W_v, b_v, W_u, b_u):
    ei = edge_index.astype(jnp.int32)
    src = ei[0]
    dst = ei[1]
    We_e = W_e[:D_E_IN]
    We_r = W_e[D_E_IN:D_E_IN + D]
    We_s = W_e[D_E_IN + D:D_E_IN + 2 * D]
    We_u = W_e[D_E_IN + 2 * D:]
    u2 = u.reshape(1, D)
    be2 = b_e.reshape(1, D)

    Xr, Xs, ec = _proj(x, We_r, We_s, u2, We_u, be2)
    A = _edge_mm(edge_attr.T, We_e, ec)
    zeros = jnp.zeros((N_PAD, D), jnp.float32)
    e_new, aggr2 = _sc_edges(A, dst, src, Xr, Xs, zeros)
    v_new, u_new2 = _node(aggr2, x, u2, W_v, b_v, W_u, b_u)
    return (e_new, v_new, u_new2.reshape(D))


# speedup vs baseline: 7.3327x; 1.0052x over previous
import functools

import jax
import jax.numpy as jnp
import numpy as np
from jax import lax
from jax.experimental import pallas as pl
from jax.experimental.pallas import tpu as pltpu
from jax.experimental.pallas import tpu_sc as plsc

N_NODES = 10000
N_EDGES = 320000
D = 128
D_E_IN = 16

NC = 2
NS = 16
NW = NC * NS
CH = 64
TCHUNK = N_EDGES // CH
CBASE = TCHUNK // NW
EXTRA = TCHUNK - CBASE * NW
N_PAD = 10112
RPT = N_PAD // NS

EHALF = N_EDGES // 2
W_CH = CH // 2


def _proj_body(x_ref, wr_ref, ws_ref, u_ref, weu_ref, be_ref,
               xr_ref, xs_ref, ec_ref):
    xr_ref[...] = jnp.dot(x_ref[...], wr_ref[...],
                          preferred_element_type=jnp.float32)
    xs_ref[...] = jnp.dot(x_ref[...], ws_ref[...],
                          preferred_element_type=jnp.float32)
    ec_ref[...] = jnp.dot(u_ref[...], weu_ref[...],
                          preferred_element_type=jnp.float32) + be_ref[...]


def _proj(x, We_r, We_s, u2, We_u, be2):
    return pl.pallas_call(
        _proj_body,
        out_shape=(
            jax.ShapeDtypeStruct((N_NODES, D), jnp.float32),
            jax.ShapeDtypeStruct((N_NODES, D), jnp.float32),
            jax.ShapeDtypeStruct((1, D), jnp.float32),
        ),
    )(x, We_r, We_s, u2, We_u, be2)


_EB = 3200


def _edge_body(ea0_ref, ea1_ref, w_ref, ec_ref, o_ref):
    dn = (((0,), (0,)), ((), ()))
    a0 = lax.dot_general(ea0_ref[...], w_ref[...], dn,
                         preferred_element_type=jnp.float32) + ec_ref[...]
    a1 = lax.dot_general(ea1_ref[...], w_ref[...], dn,
                         preferred_element_type=jnp.float32) + ec_ref[...]
    b0 = lax.bitcast_convert_type(a0.astype(jnp.bfloat16),
                                  jnp.int16).astype(jnp.int32)
    b1 = lax.bitcast_convert_type(a1.astype(jnp.bfloat16),
                                  jnp.int16).astype(jnp.int32)
    o_ref[...] = (b0 & jnp.int32(0xFFFF)) | (b1 << 16)


def _edge_mm(edge_attr_t, We_e, ec):
    nh = EHALF // _EB
    return pl.pallas_call(
        _edge_body,
        grid=(nh,),
        in_specs=[
            pl.BlockSpec((D_E_IN, _EB), lambda i: (0, i)),
            pl.BlockSpec((D_E_IN, _EB), lambda i, _nh=nh: (0, i + _nh)),
            pl.BlockSpec((D_E_IN, D), lambda i: (0, 0)),
            pl.BlockSpec((1, D), lambda i: (0, 0)),
        ],
        out_specs=pl.BlockSpec((_EB, D), lambda i: (i, 0)),
        out_shape=jax.ShapeDtypeStruct((EHALF, D), jnp.int32),
    )(edge_attr_t, edge_attr_t, We_e, ec)


class _Slot:
    def __init__(self, gd, sd, xr, xs, ab, gi, si, r, s, a, st, sc):
        self.gd = gd
        self.sd = sd
        self.xr = xr
        self.xs = xs
        self.ab = ab
        self.gi = gi
        self.si = si
        self.r = r
        self.s = s
        self.a = a
        self.st = st
        self.sc = sc


def _sc_body(a_hbm, idx_hbm, dstp_hbm, xr_hbm, xs_hbm, zeros_hbm,
             e_hbm, aggr_hbm, *rest):
    cid = lax.axis_index("c")
    sid = lax.axis_index("s")
    wid = sid * NC + cid
    nch = CBASE + (wid < EXTRA).astype(jnp.int32)

    bufs, sems, aggr_sh = rest[:10], rest[11:], rest[10]
    SLOT = tuple(_Slot(*bufs[5 * i:5 * i + 5], *sems[7 * i:7 * i + 7])
                 for i in range(2))

    pltpu.sync_copy(zeros_hbm.at[pl.ds(sid * RPT, RPT)],
                    aggr_sh.at[pl.ds(sid * RPT, RPT)])
    plsc.subcore_barrier()

    def chunk_id(k):
        return k * NW + wid

    def issue_gidx(k, S):
        pltpu.async_copy(idx_hbm.at[pl.ds(chunk_id(k) * (2 * CH), 2 * CH)],
                         S.gd, S.gi)

    def wait_gidx(S):
        pltpu.make_async_copy(idx_hbm.at[pl.ds(0, 2 * CH)], S.gd,
                              S.gi).wait()

    def issue_sidx(k, S):
        pltpu.async_copy(dstp_hbm.at[pl.ds(chunk_id(k) * CH, CH)], S.sd, S.si)

    def issue_gathers(k, S):
        pltpu.async_copy(xr_hbm.at[S.gd.at[pl.ds(0, CH)]], S.xr, S.r)
        pltpu.async_copy(xs_hbm.at[S.gd.at[pl.ds(CH, CH)]], S.xs, S.s)
        pltpu.async_copy(a_hbm.at[pl.ds(chunk_id(k) * W_CH, W_CH)], S.ab, S.a)

    def wait_in(S):
        dm = xr_hbm.at[pl.ds(0, CH)]
        pltpu.make_async_copy(dm, S.xr, S.r).wait()
        pltpu.make_async_copy(dm, S.xs, S.s).wait()
        pltpu.make_async_copy(a_hbm.at[pl.ds(0, W_CH)], S.ab, S.a).wait()

    def compute(S):
        def rowpair(a, c2):
            i1 = a + W_CH
            for j in range(D // 16):
                slc = pl.ds(16 * j, 16)
                w = S.ab[a, slc]
                lo = lax.bitcast_convert_type(w << 16, jnp.float32)
                hi = lax.bitcast_convert_type(w & jnp.int32(-65536),
                                              jnp.float32)
                S.xr[a, slc] = jnp.maximum(
                    lo + S.xr[a, slc] + S.xs[a, slc], 0.0)
                S.xr[i1, slc] = jnp.maximum(
                    hi + S.xr[i1, slc] + S.xs[i1, slc], 0.0)
            return c2

        lax.fori_loop(0, W_CH, rowpair, 0)

    def issue_out(k, S):
        pltpu.make_async_copy(dstp_hbm.at[pl.ds(0, CH)], S.sd, S.si).wait()
        base = chunk_id(k) * W_CH
        pltpu.async_copy(S.xr.at[pl.ds(0, W_CH)],
                         e_hbm.at[pl.ds(base, W_CH)], S.st)
        pltpu.async_copy(S.xr.at[pl.ds(W_CH, W_CH)],
                         e_hbm.at[pl.ds(EHALF + base, W_CH)], S.st)
        pltpu.async_copy(S.xr, aggr_sh.at[S.sd], S.sc, add=True)

    def wait_out(S):
        dm = xr_hbm.at[pl.ds(0, CH)]
        pltpu.make_async_copy(dm, S.xr, S.st).wait()
        pltpu.make_async_copy(dm, S.xr, S.sc).wait()

    def step(k, cur, nxt, first=False):
        @pl.when(k + 1 < nch)
        def _():
            if not first:
                wait_out(nxt)
            issue_sidx(k + 1, nxt)
            wait_gidx(nxt)
            issue_gathers(k + 1, nxt)

        wait_in(cur)

        @pl.when(k + 2 < nch)
        def _():
            issue_gidx(k + 2, cur)

        compute(cur)
        issue_out(k, cur)

    issue_gidx(0, SLOT[0])
    issue_sidx(0, SLOT[0])
    issue_gidx(1, SLOT[1])
    wait_gidx(SLOT[0])
    issue_gathers(0, SLOT[0])

    step(0, SLOT[0], SLOT[1], first=True)

    def pair(g, carry):
        k = 2 * g + 1
        step(k, SLOT[1], SLOT[0])
        step(k + 1, SLOT[0], SLOT[1])
        return carry

    lax.fori_loop(0, (CBASE - 1) // 2, pair, 0)
    step(CBASE - 1, SLOT[1], SLOT[0])

    @pl.when(nch > CBASE)
    def _():
        step(CBASE, SLOT[0], SLOT[1])

    wait_out(SLOT[0])
    wait_out(SLOT[1])
    plsc.subcore_barrier()
    pltpu.sync_copy(aggr_sh.at[pl.ds(sid * RPT, RPT)],
                    aggr_hbm.at[cid, pl.ds(sid * RPT, RPT)])


def _sc_edges(A32, idx_all, dst_flat, Xr, Xs, zeros):
    mesh = plsc.VectorSubcoreMesh(core_axis_name="c", subcore_axis_name="s")
    slot_bufs = []
    for _ in range(2):
        slot_bufs += [pltpu.VMEM((2 * CH,), jnp.int32),
                      pltpu.VMEM((CH,), jnp.int32),
                      pltpu.VMEM((CH, D), jnp.float32),
                      pltpu.VMEM((CH, D), jnp.float32),
                      pltpu.VMEM((W_CH, D), jnp.int32)]
    fn = functools.partial(
        pl.kernel,
        mesh=mesh,
        out_type=(
            jax.ShapeDtypeStruct((N_EDGES, D), jnp.float32),
            jax.ShapeDtypeStruct((NC, N_PAD, D), jnp.float32),
        ),
        scratch_types=slot_bufs + [
            pltpu.VMEM_SHARED((N_PAD, D), jnp.float32),
        ] + [pltpu.SemaphoreType.DMA] * 14,
    )(_sc_body)
    return fn(A32, idx_all, dst_flat, Xr, Xs, zeros)


def _node_body(ag_ref, x_ref, u_ref, wva_ref, wvx_ref, wvu_ref, bv_ref,
               wue_ref, wuv_ref, wuu_ref, bu_ref, v_ref, un_ref):
    aggr = ag_ref[0, :N_NODES] + ag_ref[1, :N_NODES]
    cv = jnp.dot(u_ref[...], wvu_ref[...],
                 preferred_element_type=jnp.float32) + bv_ref[...]
    v = jnp.maximum(
        jnp.dot(aggr, wva_ref[...], preferred_element_type=jnp.float32)
        + jnp.dot(x_ref[...], wvx_ref[...], preferred_element_type=jnp.float32)
        + cv, 0.0)
    v_ref[...] = v
    ae = jnp.sum(aggr, axis=0, keepdims=True) * (1.0 / N_EDGES)
    av = jnp.sum(v, axis=0, keepdims=True) * (1.0 / N_NODES)
    un = (jnp.dot(ae, wue_ref[...], preferred_element_type=jnp.float32)
          + jnp.dot(av, wuv_ref[...], preferred_element_type=jnp.float32)
          + jnp.dot(u_ref[...], wuu_ref[...], preferred_element_type=jnp.float32)
          + bu_ref[...])
    un_ref[...] = jnp.maximum(un, 0.0)


def _node(aggr2, x, u2, W_v, b_v, W_u, b_u):
    return pl.pallas_call(
        _node_body,
        out_shape=(
            jax.ShapeDtypeStruct((N_NODES, D), jnp.float32),
            jax.ShapeDtypeStruct((1, D), jnp.float32),
        ),
    )(aggr2, x, u2, W_v[:D], W_v[D:2 * D], W_v[2 * D:3 * D],
      b_v.reshape(1, D), W_u[:D], W_u[D:2 * D], W_u[2 * D:3 * D],
      b_u.reshape(1, D))


def kernel(x, edge_attr, u, edge_index, W_e, b_e, W_v, b_v, W_u, b_u):
    ei = edge_index.astype(jnp.int32)
    src = ei[0]
    dst = ei[1]
    We_e = W_e[:D_E_IN]
    We_r = W_e[D_E_IN:D_E_IN + D]
    We_s = W_e[D_E_IN + D:D_E_IN + 2 * D]
    We_u = W_e[D_E_IN + 2 * D:]
    u2 = u.reshape(1, D)
    be2 = b_e.reshape(1, D)

    Xr, Xs, ec = _proj(x, We_r, We_s, u2, We_u, be2)
    A32 = _edge_mm(edge_attr.T, We_e, ec)
    dst_p = dst.reshape(2, TCHUNK, W_CH).transpose(1, 0, 2).reshape(TCHUNK, CH)
    src_p = src.reshape(2, TCHUNK, W_CH).transpose(1, 0, 2).reshape(TCHUNK, CH)
    idx_all = jnp.concatenate([dst_p, src_p], axis=1).reshape(-1)
    dst_flat = dst_p.reshape(-1)
    zeros = jnp.zeros((N_PAD, D), jnp.float32)
    e_new, aggr2 = _sc_edges(A32, idx_all, dst_flat, Xr, Xs, zeros)
    v_new, u_new2 = _node(aggr2, x, u2, W_v, b_v, W_u, b_u)
    return (e_new, v_new, u_new2.reshape(D))


# speedup vs baseline: 8.1461x; 1.1109x over previous
import functools

import jax
import jax.numpy as jnp
import numpy as np
from jax import lax
from jax.experimental import pallas as pl
from jax.experimental.pallas import tpu as pltpu
from jax.experimental.pallas import tpu_sc as plsc

N_NODES = 10000
N_EDGES = 320000
D = 128
D_E_IN = 16

NC = 2
NS = 16
NW = NC * NS
CH = 64
TCHUNK = N_EDGES // CH
CBASE = TCHUNK // NW
EXTRA = TCHUNK - CBASE * NW
N_PAD = 10112
RPT = N_PAD // NS

EHALF = N_EDGES // 2
W_CH = CH // 2


def _proj_body(x_ref, wr_ref, ws_ref, u_ref, weu_ref, be_ref,
               xr_ref, xs_ref, ec_ref):
    xr_ref[...] = jnp.dot(x_ref[...], wr_ref[...],
                          preferred_element_type=jnp.float32)
    xs_ref[...] = jnp.dot(x_ref[...], ws_ref[...],
                          preferred_element_type=jnp.float32)
    ec_ref[...] = jnp.dot(u_ref[...], weu_ref[...],
                          preferred_element_type=jnp.float32) + be_ref[...]


def _proj(x, We_r, We_s, u2, We_u, be2):
    return pl.pallas_call(
        _proj_body,
        out_shape=(
            jax.ShapeDtypeStruct((N_NODES, D), jnp.float32),
            jax.ShapeDtypeStruct((N_NODES, D), jnp.float32),
            jax.ShapeDtypeStruct((1, D), jnp.float32),
        ),
    )(x, We_r, We_s, u2, We_u, be2)


_EB = 3200


def _edge_body(ea0_ref, ea1_ref, w_ref, ec_ref, o_ref):
    dn = (((0,), (0,)), ((), ()))
    a0 = lax.dot_general(ea0_ref[...], w_ref[...], dn,
                         preferred_element_type=jnp.float32) + ec_ref[...]
    a1 = lax.dot_general(ea1_ref[...], w_ref[...], dn,
                         preferred_element_type=jnp.float32) + ec_ref[...]
    b0 = lax.bitcast_convert_type(a0.astype(jnp.bfloat16),
                                  jnp.int16).astype(jnp.int32)
    b1 = lax.bitcast_convert_type(a1.astype(jnp.bfloat16),
                                  jnp.int16).astype(jnp.int32)
    o_ref[...] = (b0 & jnp.int32(0xFFFF)) | (b1 << 16)


def _edge_mm(edge_attr_t, We_e, ec):
    nh = EHALF // _EB
    return pl.pallas_call(
        _edge_body,
        grid=(nh,),
        in_specs=[
            pl.BlockSpec((D_E_IN, _EB), lambda i: (0, i)),
            pl.BlockSpec((D_E_IN, _EB), lambda i, _nh=nh: (0, i + _nh)),
            pl.BlockSpec((D_E_IN, D), lambda i: (0, 0)),
            pl.BlockSpec((1, D), lambda i: (0, 0)),
        ],
        out_specs=pl.BlockSpec((_EB, D), lambda i: (i, 0)),
        out_shape=jax.ShapeDtypeStruct((EHALF, D), jnp.int32),
    )(edge_attr_t, edge_attr_t, We_e, ec)


class _Slot:
    def __init__(self, gd, sd, xr, xs, ab, gi, si, r, s, a, st, sc):
        self.gd = gd
        self.sd = sd
        self.xr = xr
        self.xs = xs
        self.ab = ab
        self.gi = gi
        self.si = si
        self.r = r
        self.s = s
        self.a = a
        self.st = st
        self.sc = sc


def _sc_body(a_hbm, dst_hbm, src_hbm, xr_hbm, xs_hbm, zeros_hbm,
             e_hbm, aggr_hbm, *rest):
    cid = lax.axis_index("c")
    sid = lax.axis_index("s")
    wid = sid * NC + cid
    nch = CBASE + (wid < EXTRA).astype(jnp.int32)

    bufs, sems, aggr_sh = rest[:10], rest[11:], rest[10]
    SLOT = tuple(_Slot(*bufs[5 * i:5 * i + 5], *sems[7 * i:7 * i + 7])
                 for i in range(2))

    pltpu.sync_copy(zeros_hbm.at[pl.ds(sid * RPT, RPT)],
                    aggr_sh.at[pl.ds(sid * RPT, RPT)])
    plsc.subcore_barrier()

    def chunk_id(k):
        return k * NW + wid

    def issue_gidx(k, S):
        base = chunk_id(k) * W_CH
        pltpu.async_copy(dst_hbm.at[pl.ds(base, W_CH)],
                         S.gd.at[pl.ds(0, W_CH)], S.gi)
        pltpu.async_copy(dst_hbm.at[pl.ds(EHALF + base, W_CH)],
                         S.gd.at[pl.ds(W_CH, W_CH)], S.gi)
        pltpu.async_copy(src_hbm.at[pl.ds(base, W_CH)],
                         S.gd.at[pl.ds(CH, W_CH)], S.gi)
        pltpu.async_copy(src_hbm.at[pl.ds(EHALF + base, W_CH)],
                         S.gd.at[pl.ds(CH + W_CH, W_CH)], S.gi)

    def wait_gidx(S):
        dm = dst_hbm.at[pl.ds(0, W_CH)]
        for q in range(4):
            pltpu.make_async_copy(dm, S.gd.at[pl.ds(q * W_CH, W_CH)],
                                  S.gi).wait()

    def issue_sidx(k, S):
        base = chunk_id(k) * W_CH
        pltpu.async_copy(dst_hbm.at[pl.ds(base, W_CH)],
                         S.sd.at[pl.ds(0, W_CH)], S.si)
        pltpu.async_copy(dst_hbm.at[pl.ds(EHALF + base, W_CH)],
                         S.sd.at[pl.ds(W_CH, W_CH)], S.si)

    def issue_gathers(k, S):
        pltpu.async_copy(xr_hbm.at[S.gd.at[pl.ds(0, CH)]], S.xr, S.r)
        pltpu.async_copy(xs_hbm.at[S.gd.at[pl.ds(CH, CH)]], S.xs, S.s)
        pltpu.async_copy(a_hbm.at[pl.ds(chunk_id(k) * W_CH, W_CH)], S.ab, S.a)

    def wait_in(S):
        dm = xr_hbm.at[pl.ds(0, CH)]
        pltpu.make_async_copy(dm, S.xr, S.r).wait()
        pltpu.make_async_copy(dm, S.xs, S.s).wait()
        pltpu.make_async_copy(a_hbm.at[pl.ds(0, W_CH)], S.ab, S.a).wait()

    def compute(S):
        def rowpair(a, c2):
            i1 = a + W_CH
            for j in range(D // 16):
                slc = pl.ds(16 * j, 16)
                w = S.ab[a, slc]
                lo = lax.bitcast_convert_type(w << 16, jnp.float32)
                hi = lax.bitcast_convert_type(w & jnp.int32(-65536),
                                              jnp.float32)
                S.xr[a, slc] = jnp.maximum(
                    lo + S.xr[a, slc] + S.xs[a, slc], 0.0)
                S.xr[i1, slc] = jnp.maximum(
                    hi + S.xr[i1, slc] + S.xs[i1, slc], 0.0)
            return c2

        lax.fori_loop(0, W_CH, rowpair, 0)

    def issue_out(k, S):
        dm = dst_hbm.at[pl.ds(0, W_CH)]
        pltpu.make_async_copy(dm, S.sd.at[pl.ds(0, W_CH)], S.si).wait()
        pltpu.make_async_copy(dm, S.sd.at[pl.ds(W_CH, W_CH)], S.si).wait()
        base = chunk_id(k) * W_CH
        pltpu.async_copy(S.xr.at[pl.ds(0, W_CH)],
                         e_hbm.at[pl.ds(base, W_CH)], S.st)
        pltpu.async_copy(S.xr.at[pl.ds(W_CH, W_CH)],
                         e_hbm.at[pl.ds(EHALF + base, W_CH)], S.st)
        pltpu.async_copy(S.xr, aggr_sh.at[S.sd], S.sc, add=True)

    def wait_out(S):
        dm = xr_hbm.at[pl.ds(0, CH)]
        pltpu.make_async_copy(dm, S.xr, S.st).wait()
        pltpu.make_async_copy(dm, S.xr, S.sc).wait()

    def step(k, cur, nxt, first=False):
        @pl.when(k + 1 < nch)
        def _():
            if not first:
                wait_out(nxt)
            issue_sidx(k + 1, nxt)
            wait_gidx(nxt)
            issue_gathers(k + 1, nxt)

        wait_in(cur)

        @pl.when(k + 2 < nch)
        def _():
            issue_gidx(k + 2, cur)

        compute(cur)
        issue_out(k, cur)

    issue_gidx(0, SLOT[0])
    issue_sidx(0, SLOT[0])
    issue_gidx(1, SLOT[1])
    wait_gidx(SLOT[0])
    issue_gathers(0, SLOT[0])

    step(0, SLOT[0], SLOT[1], first=True)

    def pair(g, carry):
        k = 2 * g + 1
        step(k, SLOT[1], SLOT[0])
        step(k + 1, SLOT[0], SLOT[1])
        return carry

    lax.fori_loop(0, (CBASE - 1) // 2, pair, 0)
    step(CBASE - 1, SLOT[1], SLOT[0])

    @pl.when(nch > CBASE)
    def _():
        step(CBASE, SLOT[0], SLOT[1])

    wait_out(SLOT[0])
    wait_out(SLOT[1])
    plsc.subcore_barrier()
    pltpu.sync_copy(aggr_sh.at[pl.ds(sid * RPT, RPT)],
                    aggr_hbm.at[cid, pl.ds(sid * RPT, RPT)])


def _sc_edges(A32, dst, src, Xr, Xs, zeros):
    mesh = plsc.VectorSubcoreMesh(core_axis_name="c", subcore_axis_name="s")
    slot_bufs = []
    for _ in range(2):
        slot_bufs += [pltpu.VMEM((2 * CH,), jnp.int32),
                      pltpu.VMEM((CH,), jnp.int32),
                      pltpu.VMEM((CH, D), jnp.float32),
                      pltpu.VMEM((CH, D), jnp.float32),
                      pltpu.VMEM((W_CH, D), jnp.int32)]
    fn = functools.partial(
        pl.kernel,
        mesh=mesh,
        out_type=(
            jax.ShapeDtypeStruct((N_EDGES, D), jnp.float32),
            jax.ShapeDtypeStruct((NC, N_PAD, D), jnp.float32),
        ),
        scratch_types=slot_bufs + [
            pltpu.VMEM_SHARED((N_PAD, D), jnp.float32),
        ] + [pltpu.SemaphoreType.DMA] * 14,
    )(_sc_body)
    return fn(A32, dst, src, Xr, Xs, zeros)


def _node_body(ag_ref, x_ref, u_ref, wva_ref, wvx_ref, wvu_ref, bv_ref,
               wue_ref, wuv_ref, wuu_ref, bu_ref, v_ref, un_ref):
    aggr = ag_ref[0, :N_NODES] + ag_ref[1, :N_NODES]
    cv = jnp.dot(u_ref[...], wvu_ref[...],
                 preferred_element_type=jnp.float32) + bv_ref[...]
    v = jnp.maximum(
        jnp.dot(aggr, wva_ref[...], preferred_element_type=jnp.float32)
        + jnp.dot(x_ref[...], wvx_ref[...], preferred_element_type=jnp.float32)
        + cv, 0.0)
    v_ref[...] = v
    ae = jnp.sum(aggr, axis=0, keepdims=True) * (1.0 / N_EDGES)
    av = jnp.sum(v, axis=0, keepdims=True) * (1.0 / N_NODES)
    un = (jnp.dot(ae, wue_ref[...], preferred_element_type=jnp.float32)
          + jnp.dot(av, wuv_ref[...], preferred_element_type=jnp.float32)
          + jnp.dot(u_ref[...], wuu_ref[...], preferred_element_type=jnp.float32)
          + bu_ref[...])
    un_ref[...] = jnp.maximum(un, 0.0)


def _node(aggr2, x, u2, W_v, b_v, W_u, b_u):
    return pl.pallas_call(
        _node_body,
        out_shape=(
            jax.ShapeDtypeStruct((N_NODES, D), jnp.float32),
            jax.ShapeDtypeStruct((1, D), jnp.float32),
        ),
    )(aggr2, x, u2, W_v[:D], W_v[D:2 * D], W_v[2 * D:3 * D],
      b_v.reshape(1, D), W_u[:D], W_u[D:2 * D], W_u[2 * D:3 * D],
      b_u.reshape(1, D))


def kernel(x, edge_attr, u, edge_index, W_e, b_e, W_v, b_v, W_u, b_u):
    ei = edge_index.astype(jnp.int32)
    src = ei[0]
    dst = ei[1]
    We_e = W_e[:D_E_IN]
    We_r = W_e[D_E_IN:D_E_IN + D]
    We_s = W_e[D_E_IN + D:D_E_IN + 2 * D]
    We_u = W_e[D_E_IN + 2 * D:]
    u2 = u.reshape(1, D)
    be2 = b_e.reshape(1, D)

    Xr, Xs, ec = _proj(x, We_r, We_s, u2, We_u, be2)
    A32 = _edge_mm(edge_attr.T, We_e, ec)
    zeros = jnp.zeros((N_PAD, D), jnp.float32)
    e_new, aggr2 = _sc_edges(A32, dst, src, Xr, Xs, zeros)
    v_new, u_new2 = _node(aggr2, x, u2, W_v, b_v, W_u, b_u)
    return (e_new, v_new, u_new2.reshape(D))


# speedup vs baseline: 8.1633x; 1.0021x over previous
import functools

import jax
import jax.numpy as jnp
from jax import lax
from jax.experimental import pallas as pl
from jax.experimental.pallas import tpu as pltpu
from jax.experimental.pallas import tpu_sc as plsc

N_NODES = 10000
N_EDGES = 320000
D = 128
D_E_IN = 16

NC = 2
NS = 16
NW = NC * NS
CH = 64
TCHUNK = N_EDGES // CH
CBASE = TCHUNK // NW
EXTRA = TCHUNK - CBASE * NW
N_PAD = 10112
RPT = N_PAD // NS

EHALF = N_EDGES // 2
W_CH = CH // 2


def _proj_body(x_ref, wr_ref, ws_ref, u_ref, weu_ref, be_ref,
               xr_ref, xs_ref, ec_ref):
    xr_ref[...] = jnp.dot(x_ref[...], wr_ref[...],
                          preferred_element_type=jnp.float32)
    xs_ref[...] = jnp.dot(x_ref[...], ws_ref[...],
                          preferred_element_type=jnp.float32)
    ec_ref[...] = jnp.dot(u_ref[...], weu_ref[...],
                          preferred_element_type=jnp.float32) + be_ref[...]


def _proj(x, We_r, We_s, u2, We_u, be2):
    return pl.pallas_call(
        _proj_body,
        out_shape=(
            jax.ShapeDtypeStruct((N_NODES, D), jnp.float32),
            jax.ShapeDtypeStruct((N_NODES, D), jnp.float32),
            jax.ShapeDtypeStruct((1, D), jnp.float32),
        ),
    )(x, We_r, We_s, u2, We_u, be2)


_EB = 3200


def _edge_body(ea0_ref, ea1_ref, w_ref, ec_ref, o_ref):
    dn = (((0,), (0,)), ((), ()))
    a0 = lax.dot_general(ea0_ref[...], w_ref[...], dn,
                         preferred_element_type=jnp.float32) + ec_ref[...]
    a1 = lax.dot_general(ea1_ref[...], w_ref[...], dn,
                         preferred_element_type=jnp.float32) + ec_ref[...]
    b0 = lax.bitcast_convert_type(a0.astype(jnp.bfloat16),
                                  jnp.int16).astype(jnp.int32)
    b1 = lax.bitcast_convert_type(a1.astype(jnp.bfloat16),
                                  jnp.int16).astype(jnp.int32)
    o_ref[...] = (b0 & jnp.int32(0xFFFF)) | (b1 << 16)


def _edge_mm(edge_attr_t, We_e, ec):
    nh = EHALF // _EB
    return pl.pallas_call(
        _edge_body,
        grid=(nh,),
        in_specs=[
            pl.BlockSpec((D_E_IN, _EB), lambda i: (0, i)),
            pl.BlockSpec((D_E_IN, _EB), lambda i, _nh=nh: (0, i + _nh)),
            pl.BlockSpec((D_E_IN, D), lambda i: (0, 0)),
            pl.BlockSpec((1, D), lambda i: (0, 0)),
        ],
        out_specs=pl.BlockSpec((_EB, D), lambda i: (i, 0)),
        out_shape=jax.ShapeDtypeStruct((EHALF, D), jnp.int32),
    )(edge_attr_t, edge_attr_t, We_e, ec)


class _Slot:
    def __init__(self, gd, sd, xr, xs, ab, gi, si, r, s, a, st, sc):
        self.gd = gd
        self.sd = sd
        self.xr = xr
        self.xs = xs
        self.ab = ab
        self.gi = gi
        self.si = si
        self.r = r
        self.s = s
        self.a = a
        self.st = st
        self.sc = sc


def _sc_body(a_hbm, dst_hbm, src_hbm, xr_hbm, xs_hbm, zeros_hbm,
             e_hbm, aggr_hbm, *rest):
    cid = lax.axis_index("c")
    sid = lax.axis_index("s")
    wid = sid * NC + cid
    nch = CBASE + (wid < EXTRA).astype(jnp.int32)

    bufs, sems, aggr_sh = rest[:10], rest[11:], rest[10]
    SLOT = tuple(_Slot(*bufs[5 * i:5 * i + 5], *sems[7 * i:7 * i + 7])
                 for i in range(2))

    pltpu.sync_copy(zeros_hbm.at[pl.ds(sid * RPT, RPT)],
                    aggr_sh.at[pl.ds(sid * RPT, RPT)])
    plsc.subcore_barrier()

    def chunk_id(k):
        return k * NW + wid

    def issue_gidx(k, S):
        base = chunk_id(k) * W_CH
        pltpu.async_copy(dst_hbm.at[pl.ds(base, W_CH)],
                         S.gd.at[pl.ds(0, W_CH)], S.gi)
        pltpu.async_copy(dst_hbm.at[pl.ds(EHALF + base, W_CH)],
                         S.gd.at[pl.ds(W_CH, W_CH)], S.gi)
        pltpu.async_copy(src_hbm.at[pl.ds(base, W_CH)],
                         S.gd.at[pl.ds(CH, W_CH)], S.gi)
        pltpu.async_copy(src_hbm.at[pl.ds(EHALF + base, W_CH)],
                         S.gd.at[pl.ds(CH + W_CH, W_CH)], S.gi)

    def wait_gidx(S):
        dm = dst_hbm.at[pl.ds(0, W_CH)]
        for q in range(4):
            pltpu.make_async_copy(dm, S.gd.at[pl.ds(q * W_CH, W_CH)],
                                  S.gi).wait()

    def issue_sidx(k, S):
        base = chunk_id(k) * W_CH
        pltpu.async_copy(dst_hbm.at[pl.ds(base, W_CH)],
                         S.sd.at[pl.ds(0, W_CH)], S.si)
        pltpu.async_copy(dst_hbm.at[pl.ds(EHALF + base, W_CH)],
                         S.sd.at[pl.ds(W_CH, W_CH)], S.si)

    def issue_gathers(k, S):
        pltpu.async_copy(xr_hbm.at[S.gd.at[pl.ds(0, CH)]], S.xr, S.r)
        pltpu.async_copy(xs_hbm.at[S.gd.at[pl.ds(CH, CH)]], S.xs, S.s)
        pltpu.async_copy(a_hbm.at[pl.ds(chunk_id(k) * W_CH, W_CH)], S.ab, S.a)

    def wait_in(S):
        dm = xr_hbm.at[pl.ds(0, CH)]
        pltpu.make_async_copy(dm, S.xr, S.r).wait()
        pltpu.make_async_copy(dm, S.xs, S.s).wait()
        pltpu.make_async_copy(a_hbm.at[pl.ds(0, W_CH)], S.ab, S.a).wait()

    def compute(S):
        def rowpair(a, c2):
            i1 = a + W_CH
            for j in range(D // 16):
                slc = pl.ds(16 * j, 16)
                w = S.ab[a, slc]
                lo = lax.bitcast_convert_type(w << 16, jnp.float32)
                hi = lax.bitcast_convert_type(w & jnp.int32(-65536),
                                              jnp.float32)
                S.xr[a, slc] = jnp.maximum(
                    lo + S.xr[a, slc] + S.xs[a, slc], 0.0)
                S.xr[i1, slc] = jnp.maximum(
                    hi + S.xr[i1, slc] + S.xs[i1, slc], 0.0)
            return c2

        lax.fori_loop(0, W_CH, rowpair, 0)

    def issue_out(k, S):
        dm = dst_hbm.at[pl.ds(0, W_CH)]
        pltpu.make_async_copy(dm, S.sd.at[pl.ds(0, W_CH)], S.si).wait()
        pltpu.make_async_copy(dm, S.sd.at[pl.ds(W_CH, W_CH)], S.si).wait()
        base = chunk_id(k) * W_CH
        pltpu.async_copy(S.xr.at[pl.ds(0, W_CH)],
                         e_hbm.at[pl.ds(base, W_CH)], S.st)
        pltpu.async_copy(S.xr.at[pl.ds(W_CH, W_CH)],
                         e_hbm.at[pl.ds(EHALF + base, W_CH)], S.st)
        pltpu.async_copy(S.xr, aggr_sh.at[S.sd], S.sc, add=True)

    def wait_out(S):
        dm = xr_hbm.at[pl.ds(0, CH)]
        pltpu.make_async_copy(dm, S.xr, S.st).wait()
        pltpu.make_async_copy(dm, S.xr, S.sc).wait()

    def step(k, cur, nxt, first=False):
        @pl.when(k + 1 < nch)
        def _():
            if not first:
                wait_out(nxt)
            issue_sidx(k + 1, nxt)
            wait_gidx(nxt)
            issue_gathers(k + 1, nxt)

        wait_in(cur)

        @pl.when(k + 2 < nch)
        def _():
            issue_gidx(k + 2, cur)

        compute(cur)
        issue_out(k, cur)

    issue_gidx(0, SLOT[0])
    issue_sidx(0, SLOT[0])
    issue_gidx(1, SLOT[1])
    wait_gidx(SLOT[0])
    issue_gathers(0, SLOT[0])

    step(0, SLOT[0], SLOT[1], first=True)

    def pair(g, carry):
        k = 2 * g + 1
        step(k, SLOT[1], SLOT[0])
        step(k + 1, SLOT[0], SLOT[1])
        return carry

    lax.fori_loop(0, (CBASE - 1) // 2, pair, 0)
    step(CBASE - 1, SLOT[1], SLOT[0])

    @pl.when(nch > CBASE)
    def _():
        step(CBASE, SLOT[0], SLOT[1])

    wait_out(SLOT[0])
    wait_out(SLOT[1])
    plsc.subcore_barrier()
    pltpu.sync_copy(aggr_sh.at[pl.ds(sid * RPT, RPT)],
                    aggr_hbm.at[cid, pl.ds(sid * RPT, RPT)])


def _sc_edges(A32, dst, src, Xr, Xs, zeros):
    mesh = plsc.VectorSubcoreMesh(core_axis_name="c", subcore_axis_name="s")
    slot_bufs = []
    for _ in range(2):
        slot_bufs += [pltpu.VMEM((2 * CH,), jnp.int32),
                      pltpu.VMEM((CH,), jnp.int32),
                      pltpu.VMEM((CH, D), jnp.float32),
                      pltpu.VMEM((CH, D), jnp.float32),
                      pltpu.VMEM((W_CH, D), jnp.int32)]
    fn = functools.partial(
        pl.kernel,
        mesh=mesh,
        out_type=(
            jax.ShapeDtypeStruct((N_EDGES, D), jnp.float32),
            jax.ShapeDtypeStruct((NC, N_PAD, D), jnp.float32),
        ),
        scratch_types=slot_bufs + [
            pltpu.VMEM_SHARED((N_PAD, D), jnp.float32),
        ] + [pltpu.SemaphoreType.DMA] * 14,
    )(_sc_body)
    return fn(A32, dst, src, Xr, Xs, zeros)


def _node_body(ag_ref, x_ref, u_ref, wva_ref, wvx_ref, wvu_ref, bv_ref,
               wue_ref, wuv_ref, wuu_ref, bu_ref, v_ref, un_ref):
    aggr = ag_ref[0, :N_NODES] + ag_ref[1, :N_NODES]
    cv = jnp.dot(u_ref[...], wvu_ref[...],
                 preferred_element_type=jnp.float32) + bv_ref[...]
    v = jnp.maximum(
        jnp.dot(aggr, wva_ref[...], preferred_element_type=jnp.float32)
        + jnp.dot(x_ref[...], wvx_ref[...], preferred_element_type=jnp.float32)
        + cv, 0.0)
    v_ref[...] = v
    ae = jnp.sum(aggr, axis=0, keepdims=True) * (1.0 / N_EDGES)
    av = jnp.sum(v, axis=0, keepdims=True) * (1.0 / N_NODES)
    un = (jnp.dot(ae, wue_ref[...], preferred_element_type=jnp.float32)
          + jnp.dot(av, wuv_ref[...], preferred_element_type=jnp.float32)
          + jnp.dot(u_ref[...], wuu_ref[...], preferred_element_type=jnp.float32)
          + bu_ref[...])
    un_ref[...] = jnp.maximum(un, 0.0)


def _node(aggr2, x, u2, W_v, b_v, W_u, b_u):
    return pl.pallas_call(
        _node_body,
        out_shape=(
            jax.ShapeDtypeStruct((N_NODES, D), jnp.float32),
            jax.ShapeDtypeStruct((1, D), jnp.float32),
        ),
    )(aggr2, x, u2, W_v[:D], W_v[D:2 * D], W_v[2 * D:3 * D],
      b_v.reshape(1, D), W_u[:D], W_u[D:2 * D], W_u[2 * D:3 * D],
      b_u.reshape(1, D))


def kernel(x, edge_attr, u, edge_index, W_e, b_e, W_v, b_v, W_u, b_u):
    ei = edge_index.astype(jnp.int32)
    src = ei[0]
    dst = ei[1]
    We_e = W_e[:D_E_IN]
    We_r = W_e[D_E_IN:D_E_IN + D]
    We_s = W_e[D_E_IN + D:D_E_IN + 2 * D]
    We_u = W_e[D_E_IN + 2 * D:]
    u2 = u.reshape(1, D)
    be2 = b_e.reshape(1, D)

    Xr, Xs, ec = _proj(x, We_r, We_s, u2, We_u, be2)
    A32 = _edge_mm(edge_attr.T, We_e, ec)
    zeros = jnp.zeros((N_PAD, D), jnp.float32)
    e_new, aggr2 = _sc_edges(A32, dst, src, Xr, Xs, zeros)
    v_new, u_new2 = _node(aggr2, x, u2, W_v, b_v, W_u, b_u)
    return (e_new, v_new, u_new2.reshape(D))
